# Initial kernel scaffold; baseline (speedup 1.0000x reference)
import jax
import jax.numpy as jnp
from jax.experimental import pallas as pl


def kernel(hidden_states, input_ids, memory_table, hash_coeffs, W, b, gate):
    raise NotImplementedError("write your pallas kernel here")


# speedup vs baseline: 2.2611x; 2.2611x over previous
import functools

import jax
import jax.numpy as jnp
from jax import lax
from jax.experimental import pallas as pl
from jax.experimental.pallas import tpu as pltpu
from jax.experimental.pallas import tpu_sc as plsc

D_MODEL = 1024
MEMORY_SIZE = 100000
NUM_HEADS = 4

_NC = 2
_NS = 16
_NW = _NC * _NS

_N_GATHER = NUM_HEADS * 4 * 2048
_ROWS_PER_W = _N_GATHER // _NW
_CHUNK = 64
_N_CHUNKS = _ROWS_PER_W // _CHUNK


def _gather_body(table_hbm, idx_hbm, out_hbm, idx_v, rows_v, sem):
    wid = lax.axis_index("s") * _NC + lax.axis_index("c")
    base = wid * _ROWS_PER_W
    pltpu.sync_copy(idx_hbm.at[pl.ds(base, _ROWS_PER_W)], idx_v)
    for i in range(_N_CHUNKS):
        pltpu.async_copy(
            table_hbm.at[idx_v.at[pl.ds(i * _CHUNK, _CHUNK)]], rows_v, sem
        ).wait()
        pltpu.sync_copy(rows_v, out_hbm.at[pl.ds(base + i * _CHUNK, _CHUNK)])


_sc_gather = functools.partial(
    pl.kernel,
    out_type=jax.ShapeDtypeStruct((_N_GATHER, D_MODEL), jnp.float32),
    mesh=plsc.VectorSubcoreMesh(core_axis_name="c", subcore_axis_name="s"),
    scratch_types=[
        pltpu.VMEM((_ROWS_PER_W,), jnp.int32),
        pltpu.VMEM((_CHUNK, D_MODEL), jnp.float32),
        pltpu.SemaphoreType.DMA,
    ],
)(_gather_body)


_BT = 512


def _mm_body(g_ref, mh_ref, w_ref, b_ref, h_ref, o_ref):
    g = g_ref[0]
    mh = mh_ref[...].astype(jnp.bfloat16)
    acc = lax.dot_general(
        mh, w_ref[...], (((1,), (1,)), ((), ())),
        preferred_element_type=jnp.float32,
    )
    o_ref[...] = (1.0 - g) * h_ref[...] + g * (acc + b_ref[...])


def _tc_project(multi, w16, b2, hidden2d, gate):
    t = multi.shape[0]
    hd = multi.shape[1]
    return pl.pallas_call(
        _mm_body,
        grid=(t // _BT,),
        in_specs=[
            pl.BlockSpec(memory_space=pltpu.SMEM),
            pl.BlockSpec((_BT, hd), lambda i: (i, 0)),
            pl.BlockSpec((D_MODEL, hd), lambda i: (0, 0)),
            pl.BlockSpec((1, D_MODEL), lambda i: (0, 0)),
            pl.BlockSpec((_BT, D_MODEL), lambda i: (i, 0)),
        ],
        out_specs=pl.BlockSpec((_BT, D_MODEL), lambda i: (i, 0)),
        out_shape=jax.ShapeDtypeStruct((t, D_MODEL), jnp.float32),
        compiler_params=pltpu.CompilerParams(
            dimension_semantics=("arbitrary",),
        ),
    )(gate, multi, w16, b2, hidden2d)


def kernel(hidden_states, input_ids, memory_table, hash_coeffs, W, b, gate):
    bsz, seq, d = hidden_states.shape
    h = hash_coeffs.shape[0]
    t = bsz * seq

    ids_f = input_ids.reshape(-1)[:, None].astype(jnp.float32)
    idx = ((ids_f * hash_coeffs[None, :]) % MEMORY_SIZE).astype(jnp.int32)
    flat_idx = idx.reshape(-1)

    multi = _sc_gather(memory_table, flat_idx)
    multi = multi.reshape(t, h * d)

    w16 = W.astype(jnp.bfloat16)
    b2 = b.reshape(1, d)
    out = _tc_project(multi, w16, b2, hidden_states.reshape(t, d), gate)
    return out.reshape(bsz, seq, d)


# speedup vs baseline: 2.2997x; 1.0171x over previous
import functools

import jax
import jax.numpy as jnp
from jax import lax
from jax.experimental import pallas as pl
from jax.experimental.pallas import tpu as pltpu
from jax.experimental.pallas import tpu_sc as plsc

D_MODEL = 1024
MEMORY_SIZE = 100000
NUM_HEADS = 4

_NC = 2
_NS = 16
_NW = _NC * _NS

_CHUNK = 32


def _make_sc_gather(n_rows):
    rows_per_w = n_rows // _NW
    n_chunks = rows_per_w // _CHUNK

    def _gather_body(table_hbm, idx_hbm, out_hbm, idx_v, rows0, rows1, gs0,
                     gs1, os0, os1):
        wid = lax.axis_index("s") * _NC + lax.axis_index("c")
        base = wid * rows_per_w
        pltpu.sync_copy(idx_hbm.at[pl.ds(base, rows_per_w)], idx_v)
        bufs, gsems, osems = (rows0, rows1), (gs0, gs1), (os0, os1)

        def _start_gather(i):
            return pltpu.async_copy(
                table_hbm.at[idx_v.at[pl.ds(i * _CHUNK, _CHUNK)]],
                bufs[i % 2], gsems[i % 2])

        out_copies = [None, None]
        gather = _start_gather(0)
        for i in range(n_chunks):
            b = i % 2
            gather.wait()
            if i + 1 < n_chunks:
                if out_copies[1 - b] is not None:
                    out_copies[1 - b].wait()
                    out_copies[1 - b] = None
                gather = _start_gather(i + 1)
            out_copies[b] = pltpu.async_copy(
                bufs[b], out_hbm.at[pl.ds(base + i * _CHUNK, _CHUNK)],
                osems[b])
        for oc in out_copies:
            if oc is not None:
                oc.wait()

    return functools.partial(
        pl.kernel,
        out_type=jax.ShapeDtypeStruct((n_rows, D_MODEL), jnp.float32),
        mesh=plsc.VectorSubcoreMesh(core_axis_name="c", subcore_axis_name="s"),
        scratch_types=[
            pltpu.VMEM((rows_per_w,), jnp.int32),
            pltpu.VMEM((_CHUNK, D_MODEL), jnp.float32),
            pltpu.VMEM((_CHUNK, D_MODEL), jnp.float32),
            pltpu.SemaphoreType.DMA,
            pltpu.SemaphoreType.DMA,
            pltpu.SemaphoreType.DMA,
            pltpu.SemaphoreType.DMA,
        ],
    )(_gather_body)


_sc_gather = _make_sc_gather(NUM_HEADS * 4 * 2048)


_BT = 512


def _mm_body(g_ref, mh_ref, w_ref, b_ref, h_ref, o_ref):
    g = g_ref[0]
    mh = mh_ref[...].astype(jnp.bfloat16)
    acc = lax.dot_general(
        mh, w_ref[...], (((1,), (1,)), ((), ())),
        preferred_element_type=jnp.float32,
    )
    o_ref[...] = (1.0 - g) * h_ref[...] + g * (acc + b_ref[...])


def _tc_project(multi, w16, b2, hidden2d, gate):
    t = multi.shape[0]
    hd = multi.shape[1]
    return pl.pallas_call(
        _mm_body,
        grid=(t // _BT,),
        in_specs=[
            pl.BlockSpec(memory_space=pltpu.SMEM),
            pl.BlockSpec((_BT, hd), lambda i: (i, 0)),
            pl.BlockSpec((D_MODEL, hd), lambda i: (0, 0)),
            pl.BlockSpec((1, D_MODEL), lambda i: (0, 0)),
            pl.BlockSpec((_BT, D_MODEL), lambda i: (i, 0)),
        ],
        out_specs=pl.BlockSpec((_BT, D_MODEL), lambda i: (i, 0)),
        out_shape=jax.ShapeDtypeStruct((t, D_MODEL), jnp.float32),
        compiler_params=pltpu.CompilerParams(
            dimension_semantics=("arbitrary",),
        ),
    )(gate, multi, w16, b2, hidden2d)


def kernel(hidden_states, input_ids, memory_table, hash_coeffs, W, b, gate):
    bsz, seq, d = hidden_states.shape
    h = hash_coeffs.shape[0]
    t = bsz * seq

    ids_f = input_ids.reshape(-1)[:, None].astype(jnp.float32)
    idx = ((ids_f * hash_coeffs[None, :]) % MEMORY_SIZE).astype(jnp.int32)
    flat_idx = idx.reshape(-1)

    multi = _sc_gather(memory_table, flat_idx)
    multi = multi.reshape(t, h * d)

    w16 = W.astype(jnp.bfloat16)
    b2 = b.reshape(1, d)
    out = _tc_project(multi, w16, b2, hidden_states.reshape(t, d), gate)
    return out.reshape(bsz, seq, d)
